# Initial kernel scaffold; baseline (speedup 1.0000x reference)
import jax
import jax.numpy as jnp
from jax.experimental import pallas as pl


def kernel(z, embedding_weight):
    raise NotImplementedError("write your pallas kernel here")


# speedup vs baseline: 18.7673x; 18.7673x over previous
import functools

import jax
import jax.numpy as jnp
from jax.experimental import pallas as pl
from jax.experimental.pallas import tpu as pltpu

_N_E = 8192
_E_DIM = 32
_K = _N_E + 1
_KPAD = 9216
_W = 384
_B = 4
_T = 256
_BT = _B * _T
_KT = 1152
_EPSN = 1e-06 / _N_E
_BETA = 0.25


def _zsq_tree(zz):
    a = ((zz[:, 8:16] + zz[:, 0:8]) + zz[:, 16:24]) + zz[:, 24:32]
    b = a[:, 0:4] + a[:, 4:8]
    c = b[:, 0:2] + b[:, 2:4]
    return c[:, 0:1] + c[:, 1:2]


def _argmin_kernel(z0_ref, embt_ref, enc0_ref):
    z0 = z0_ref[...]
    embt = embt_ref[...]
    zsq = _zsq_tree(z0[:_B] * z0[:_B])
    bsq = jnp.sum(embt * embt, axis=0, keepdims=True)
    mm = jnp.dot(z0, embt, preferred_element_type=jnp.float32)
    d0 = (zsq + bsq) - 2.0 * mm[:_B]
    kidx = jax.lax.broadcasted_iota(jnp.int32, (_B, _KPAD), 1)
    d0 = jnp.where(kidx < _K, d0, jnp.float32(1e30))
    rowmin = jnp.min(d0, axis=1, keepdims=True)
    idx = jnp.min(jnp.where(d0 == rowmin, kidx, _KPAD), axis=1, keepdims=True)
    enc0 = jnp.clip(idx, 0, _N_E - 1)
    enc0_ref[...] = jnp.broadcast_to(enc0, (_B, 128))


def _main_kernel(enc0_ref, zf_ref, emb_ref, embt_ref,
                 zq_ref, inds_ref, loss_ref, v_ref, dwin_ref):
    zf = zf_ref[...]
    zsq = _zsq_tree(zf * zf)

    ws_list = [enc0_ref[b, 0] for b in range(_B)]
    ew_list = []
    for b in range(_B):
        ew = emb_ref[pl.ds(ws_list[b], _W), :]
        ew_list.append(ew)
        ewt = jnp.transpose(ew)
        wsq = jnp.sum(ewt * ewt, axis=0, keepdims=True)
        zb = zf[b * _T:(b + 1) * _T, :]
        mmb = jnp.dot(zb, ewt, preferred_element_type=jnp.float32)
        dwin = (zsq[b * _T:(b + 1) * _T, :] + wsq) - 2.0 * mmb
        dwin_ref[:, b, :] = dwin

    rowi = jax.lax.broadcasted_iota(jnp.int32, (_B, 1), 0)
    ws_vec = jnp.full((_B, 1), ws_list[0], dtype=jnp.int32)
    for b in range(1, _B):
        ws_vec = jnp.where(rowi == b, ws_list[b], ws_vec)
    rmax = jnp.int32(_N_E - 1) - ws_vec

    lane_w = jax.lax.broadcasted_iota(jnp.int32, (_B, _W), 1)
    lane_t = jax.lax.broadcasted_iota(jnp.int32, (_B, _T), 1)

    def step(t, carry):
        r, racc = carry
        dt = dwin_ref[pl.ds(t, 1), :, :].reshape(_B, _W)
        d_here = jnp.sum(jnp.where(lane_w == r, dt, 0.0),
                         axis=1, keepdims=True)
        rn = jnp.minimum(r + 1, rmax)
        d_next = jnp.sum(jnp.where(lane_w == rn, dt, 0.0),
                         axis=1, keepdims=True)
        keep = d_here <= d_next
        r = jnp.where(keep, r, rn)
        racc = jnp.where(lane_t == t, r, racc)
        return r, racc

    r0 = jnp.zeros((_B, 1), dtype=jnp.int32)
    racc0 = jnp.zeros((_B, _T), dtype=jnp.int32)
    _, racc = jax.lax.fori_loop(1, _T, step, (r0, racc0))

    inds = racc + ws_vec
    inds_ref[...] = inds

    mx = jnp.max(inds)
    mn = jnp.min(inds)
    v_ref[0, 0] = mx - mn

    zq_parts = []
    for b in range(_B):
        rcol = racc[b].reshape(_T, 1)
        oh = (jax.lax.broadcasted_iota(jnp.int32, (_T, _W), 1)
              == rcol).astype(jnp.float32)
        zq_parts.append(jnp.dot(oh, ew_list[b],
                                preferred_element_type=jnp.float32))
    zq = jnp.concatenate(zq_parts, axis=0)
    zq_ref[...] = zf + (zq - zf)

    diff = zf - zq
    dsel = jnp.sum(diff * diff, axis=1, keepdims=True)

    embt = embt_ref[...]
    acc = jnp.zeros((_BT, 1), dtype=jnp.float32)
    for i in range(_KPAD // _KT):
        et = embt[:, i * _KT:(i + 1) * _KT]
        bsq = jnp.sum(et * et, axis=0, keepdims=True)
        mm = jnp.dot(zf, et, preferred_element_type=jnp.float32)
        d_tile = (zsq + bsq) - 2.0 * mm
        term = jnp.maximum((dsel - d_tile) + jnp.float32(_EPSN), 0.0)
        kidx = jax.lax.broadcasted_iota(jnp.int32, (_BT, _KT), 1) + i * _KT
        term = jnp.where(kidx < _K, term, 0.0)
        acc = acc + jnp.sum(term, axis=1, keepdims=True)
    total = jnp.sum(acc)
    lmean = total / jnp.float32(_B * _T * _K)
    loss_ref[0, 0] = jnp.float32(_BETA) * lmean + lmean


@jax.jit
def kernel(z, embedding_weight):
    zf = z.reshape(_BT, _E_DIM)
    z0 = jnp.pad(z[:, 0, :], ((0, _T - _B), (0, 0)))
    embp = jnp.pad(embedding_weight, ((0, _KPAD - _K), (0, 0)))
    embt = embp.T

    enc0 = pl.pallas_call(
        _argmin_kernel,
        out_shape=jax.ShapeDtypeStruct((_B, 128), jnp.int32),
    )(z0, embt)

    zq, inds, loss, v = pl.pallas_call(
        _main_kernel,
        in_specs=[
            pl.BlockSpec(memory_space=pltpu.SMEM),
            pl.BlockSpec(memory_space=pltpu.VMEM),
            pl.BlockSpec(memory_space=pltpu.VMEM),
            pl.BlockSpec(memory_space=pltpu.VMEM),
        ],
        out_shape=(
            jax.ShapeDtypeStruct((_BT, _E_DIM), jnp.float32),
            jax.ShapeDtypeStruct((_B, _T), jnp.int32),
            jax.ShapeDtypeStruct((1, 1), jnp.float32),
            jax.ShapeDtypeStruct((1, 1), jnp.int32),
        ),
        out_specs=(
            pl.BlockSpec(memory_space=pltpu.VMEM),
            pl.BlockSpec(memory_space=pltpu.VMEM),
            pl.BlockSpec(memory_space=pltpu.SMEM),
            pl.BlockSpec(memory_space=pltpu.SMEM),
        ),
        scratch_shapes=[pltpu.VMEM((_T, _B, _W), jnp.float32)],
    )(enc0[:, :1], zf, embp, embt)

    return (zq.reshape(z.shape), loss.reshape(()), inds, v.reshape(()))
